# baseline (speedup 1.0000x reference)
import dataclasses
import functools

import jax
import jax.numpy as jnp
import numpy as np
from jax import lax
from jax.experimental import pallas as pl
from jax.experimental.pallas import tpu as pltpu
from jax.experimental.pallas import tpu_sc as plsc

EPS = 1e-06
CLAMP = 50.0
KNN = 16
UP = 2
HI = lax.Precision.HIGHEST

_NW = 32
_GRP = 64


def _mm(a, b):
    return jnp.dot(a, b, precision=HI, preferred_element_type=jnp.float32)


def _mmd(a, b):
    return jnp.dot(a, b, precision=lax.Precision.DEFAULT,
                   preferred_element_type=jnp.float32)


def _sc_compiler_params():
    cp = pltpu.CompilerParams()
    if "needs_layout_passes" in pltpu.CompilerParams.__dataclass_fields__:
        cp = dataclasses.replace(cp, needs_layout_passes=False)
    if "use_tc_tiling_on_sc" in pltpu.CompilerParams.__dataclass_fields__:
        cp = dataclasses.replace(cp, use_tc_tiling_on_sc=True)
    return cp


def _proj_body(vf_ref, wq_ref, wk_ref, wu_ref, r_ref,
               qf_ref, kf_ref, uf_ref, qn_ref, kn_ref):
    vb = vf_ref[...]
    r = r_ref[...]
    q = _mmd(vb, wq_ref[...])
    k = _mmd(vb, wk_ref[...])
    u = _mmd(vb, wu_ref[...])
    qf_ref[...] = q
    kf_ref[...] = k
    uf_ref[...] = u
    qn_ref[...] = jnp.mean(jnp.sqrt(_mm(q * q, r)), axis=1, keepdims=True)
    kn_ref[...] = jnp.mean(jnp.sqrt(_mm(k * k, r)), axis=1, keepdims=True)


def _project(vf, wq3t, wk3t, wu3t, r_mat, tn=512):
    m, d = vf.shape
    c = d // 3
    grid = (m // tn,)
    f32 = jnp.float32
    return pl.pallas_call(
        _proj_body,
        grid=grid,
        in_specs=[
            pl.BlockSpec((tn, d), lambda i: (i, 0)),
            pl.BlockSpec((d, d), lambda i: (0, 0)),
            pl.BlockSpec((d, d), lambda i: (0, 0)),
            pl.BlockSpec((d, d), lambda i: (0, 0)),
            pl.BlockSpec((d, c), lambda i: (0, 0)),
        ],
        out_specs=[
            pl.BlockSpec((tn, d), lambda i: (i, 0)),
            pl.BlockSpec((tn, d), lambda i: (i, 0)),
            pl.BlockSpec((tn, d), lambda i: (i, 0)),
            pl.BlockSpec((tn, 1), lambda i: (i, 0)),
            pl.BlockSpec((tn, 1), lambda i: (i, 0)),
        ],
        out_shape=[
            jax.ShapeDtypeStruct((m, d), f32),
            jax.ShapeDtypeStruct((m, d), f32),
            jax.ShapeDtypeStruct((m, d), f32),
            jax.ShapeDtypeStruct((m, 1), f32),
            jax.ShapeDtypeStruct((m, 1), f32),
        ],
    )(vf, wq3t, wk3t, wu3t, r_mat)


def _knn_body(xr_ref, xct_ref, knt_ref, idx_ref, ssd_ref, knb_ref, *, n, tn):
    b = pl.program_id(0)
    i = pl.program_id(1)
    xr = xr_ref[...]
    xc = xct_ref[0]
    knrow = knt_ref[0]
    sqr = jnp.sum(xr * xr, axis=1, keepdims=True)
    sqc = jnp.sum(xc * xc, axis=0, keepdims=True)
    dotm = _mmd(xr, xc)
    d2 = jnp.maximum(sqr + sqc - 2.0 * dotm, 0.0)
    col = lax.broadcasted_iota(jnp.int32, (tn, n), 1)
    row = i * tn + lax.broadcasted_iota(jnp.int32, (tn, n), 0)
    big = jnp.int32(0x7FFFFFFF)
    key = ((lax.bitcast_convert_type(d2, jnp.int32) + jnp.int32(0x800))
           & jnp.int32(~0xFFF)) | col
    key = jnp.where(col == row, big, key)
    cols, ssds, kns = [], [], []
    for _ in range(KNN):
        kmin = jnp.min(key, axis=1, keepdims=True)
        eq = key == kmin
        kns.append(jnp.sum(jnp.where(eq, knrow, 0.0), axis=1, keepdims=True))
        key = jnp.where(eq, big, key)
        cols.append(kmin & jnp.int32(0xFFF))
        ssds.append(lax.bitcast_convert_type(kmin & jnp.int32(~0xFFF),
                                             jnp.float32))
    idx_ref[...] = jnp.concatenate(cols, axis=1) + b * n
    ssd_ref[...] = jnp.concatenate(ssds, axis=1)
    knb_ref[...] = jnp.concatenate(kns, axis=1)


def _knn(xp, xpt, knt, tn=256):
    bsz, _, n = xpt.shape
    m = bsz * n
    grid = (bsz, n // tn)
    body = functools.partial(_knn_body, n=n, tn=tn)
    f32 = jnp.float32
    return pl.pallas_call(
        body,
        grid=grid,
        in_specs=[
            pl.BlockSpec((tn, 8), lambda b, i: (b * (n // tn) + i, 0)),
            pl.BlockSpec((1, 8, n), lambda b, i: (b, 0, 0)),
            pl.BlockSpec((1, 1, n), lambda b, i: (b, 0, 0)),
        ],
        out_specs=[
            pl.BlockSpec((tn, KNN), lambda b, i: (b * (n // tn) + i, 0)),
            pl.BlockSpec((tn, KNN), lambda b, i: (b * (n // tn) + i, 0)),
            pl.BlockSpec((tn, KNN), lambda b, i: (b * (n // tn) + i, 0)),
        ],
        out_shape=[
            jax.ShapeDtypeStruct((m, KNN), jnp.int32),
            jax.ShapeDtypeStruct((m, KNN), f32),
            jax.ShapeDtypeStruct((m, KNN), f32),
        ],
    )(xp, xpt, knt)


def _sc1_body(q_hbm, k_hbm, idx_hbm, dot_hbm,
              idx_v, q_v, kr0_v, kr1_v, dot_v, sem0, sem1):
    c = lax.axis_index("c")
    s = lax.axis_index("s")
    wid = s * 2 + c
    per_w = q_hbm.shape[0] // _NW
    base = wid * per_w
    lanes = lax.iota(jnp.int32, 16)
    last = _GRP - 1

    def compute(p, kr_v):
        qv = [q_v[p, pl.ds(i * 16, 16)] for i in range(24)]
        dotvec = jnp.zeros((16,), jnp.float32)
        for j in range(KNN):
            acc = qv[0] * kr_v[j, pl.ds(0, 16)]
            for i in range(1, 24):
                acc = acc + qv[i] * kr_v[j, pl.ds(i * 16, 16)]
            dsc = jnp.sum(acc)
            dotvec = jnp.where(lanes == j, jnp.full((16,), dsc), dotvec)
        dot_v[p, :] = dotvec

    @pl.loop(0, per_w // _GRP)
    def _grp(g):
        gb = base + g * _GRP
        pltpu.sync_copy(idx_hbm.at[pl.ds(gb, _GRP)], idx_v)
        pltpu.sync_copy(q_hbm.at[pl.ds(gb, _GRP)], q_v)
        pltpu.async_copy(k_hbm.at[idx_v.at[0]], kr0_v, sem0)

        @pl.loop(0, _GRP // 2)
        def _pt(t):
            p0 = 2 * t
            pltpu.async_copy(k_hbm.at[idx_v.at[p0 + 1]], kr1_v, sem1)
            pltpu.make_async_copy(k_hbm.at[idx_v.at[p0]], kr0_v, sem0).wait()
            compute(p0, kr0_v)
            nxt = jnp.minimum(p0 + 2, last)
            pltpu.async_copy(k_hbm.at[idx_v.at[nxt]], kr0_v, sem0)
            pltpu.make_async_copy(k_hbm.at[idx_v.at[p0 + 1]], kr1_v,
                                  sem1).wait()
            compute(p0 + 1, kr1_v)

        pltpu.make_async_copy(k_hbm.at[idx_v.at[last]], kr0_v, sem0).wait()
        pltpu.sync_copy(dot_v, dot_hbm.at[pl.ds(gb, _GRP)])


def _sc_pass1(qf, kf, idxg):
    m, d = qf.shape
    f32 = jnp.float32
    mesh = plsc.VectorSubcoreMesh(core_axis_name="c", subcore_axis_name="s")
    fn = pl.kernel(
        _sc1_body,
        mesh=mesh,
        out_type=jax.ShapeDtypeStruct((m, KNN), f32),
        scratch_types=[
            pltpu.VMEM((_GRP, KNN), jnp.int32),
            pltpu.VMEM((_GRP, d), f32),
            pltpu.VMEM((KNN, d), f32),
            pltpu.VMEM((KNN, d), f32),
            pltpu.VMEM((_GRP, KNN), f32),
            pltpu.SemaphoreType.DMA,
            pltpu.SemaphoreType.DMA,
        ],
        compiler_params=_sc_compiler_params(),
    )
    return fn(qf, kf, idxg)


def _mlp_body(qn_ref, knb_ref, dot_ref, ssd_ref,
              w0_ref, k1kn_ref, k1dot_ref, k1dist_ref, b1_ref,
              k2_ref, b2_ref, k3_ref, b3_ref, attn_ref):
    qn = qn_ref[...]
    knb = knb_ref[...]
    dot = dot_ref[...] * (1.0 / 128.0)
    dist = jnp.sqrt(ssd_ref[...])

    def silu(t):
        return t * (1.0 / (1.0 + jnp.exp(-t)))

    h1 = (_mmd(qn, w0_ref[...]) + _mmd(knb, k1kn_ref[...])
          + _mmd(dot, k1dot_ref[...]) + _mmd(dist, k1dist_ref[...])
          + b1_ref[...])
    h1 = silu(h1)
    h2 = silu(_mmd(h1, k2_ref[...]) + b2_ref[...])
    logits = _mmd(h2, k3_ref[...]) + b3_ref[...]
    logits = jnp.clip(logits, -10.0, 10.0)
    mx = jnp.max(logits, axis=1, keepdims=True)
    e = jnp.exp(logits - mx)
    attn_ref[...] = e / jnp.sum(e, axis=1, keepdims=True)


def _mlp(qn, knb, dot16, ssd16, w0t, k1kn, k1dot, k1dist, b1t, k2, b2t, k3,
         b3t, tn=512):
    m = qn.shape[0]
    h16 = k2.shape[0]
    grid = (m // tn,)
    return pl.pallas_call(
        _mlp_body,
        grid=grid,
        in_specs=[
            pl.BlockSpec((tn, 1), lambda i: (i, 0)),
            pl.BlockSpec((tn, KNN), lambda i: (i, 0)),
            pl.BlockSpec((tn, KNN), lambda i: (i, 0)),
            pl.BlockSpec((tn, KNN), lambda i: (i, 0)),
            pl.BlockSpec((1, h16), lambda i: (0, 0)),
            pl.BlockSpec((KNN, h16), lambda i: (0, 0)),
            pl.BlockSpec((KNN, h16), lambda i: (0, 0)),
            pl.BlockSpec((KNN, h16), lambda i: (0, 0)),
            pl.BlockSpec((1, h16), lambda i: (0, 0)),
            pl.BlockSpec((h16, h16), lambda i: (0, 0)),
            pl.BlockSpec((1, h16), lambda i: (0, 0)),
            pl.BlockSpec((h16, KNN), lambda i: (0, 0)),
            pl.BlockSpec((1, KNN), lambda i: (0, 0)),
        ],
        out_specs=pl.BlockSpec((tn, KNN), lambda i: (i, 0)),
        out_shape=jax.ShapeDtypeStruct((m, KNN), jnp.float32),
    )(qn, knb, dot16, ssd16, w0t, k1kn, k1dot, k1dist, b1t, k2, b2t, k3, b3t)


def _sc2_body(u_hbm, idx_hbm, attn_hbm, msg_hbm,
              idx_v, attn_v, ur0_v, ur1_v, msg_v, sem0, sem1):
    c = lax.axis_index("c")
    s = lax.axis_index("s")
    wid = s * 2 + c
    per_w = msg_hbm.shape[0] // _NW
    base = wid * per_w
    last = _GRP - 1

    def compute(p, ur_v):
        arow = attn_v[p, :]
        accs = [jnp.zeros((16,), jnp.float32) for _ in range(24)]
        for j in range(KNN):
            wv = jnp.full((16,), arow[j])
            for i in range(24):
                accs[i] = accs[i] + wv * ur_v[j, pl.ds(i * 16, 16)]
        for i in range(24):
            msg_v[p, pl.ds(i * 16, 16)] = accs[i]

    @pl.loop(0, per_w // _GRP)
    def _grp(g):
        gb = base + g * _GRP
        pltpu.sync_copy(idx_hbm.at[pl.ds(gb, _GRP)], idx_v)
        pltpu.sync_copy(attn_hbm.at[pl.ds(gb, _GRP)], attn_v)
        pltpu.async_copy(u_hbm.at[idx_v.at[0]], ur0_v, sem0)

        @pl.loop(0, _GRP // 2)
        def _pt(t):
            p0 = 2 * t
            pltpu.async_copy(u_hbm.at[idx_v.at[p0 + 1]], ur1_v, sem1)
            pltpu.make_async_copy(u_hbm.at[idx_v.at[p0]], ur0_v, sem0).wait()
            compute(p0, ur0_v)
            nxt = jnp.minimum(p0 + 2, last)
            pltpu.async_copy(u_hbm.at[idx_v.at[nxt]], ur0_v, sem0)
            pltpu.make_async_copy(u_hbm.at[idx_v.at[p0 + 1]], ur1_v,
                                  sem1).wait()
            compute(p0 + 1, ur1_v)

        pltpu.make_async_copy(u_hbm.at[idx_v.at[last]], ur0_v, sem0).wait()
        pltpu.sync_copy(msg_v, msg_hbm.at[pl.ds(gb, _GRP)])


def _sc_pass2(uf, idxg, attn):
    m, d = uf.shape
    f32 = jnp.float32
    mesh = plsc.VectorSubcoreMesh(core_axis_name="c", subcore_axis_name="s")
    fn = pl.kernel(
        _sc2_body,
        mesh=mesh,
        out_type=jax.ShapeDtypeStruct((m, d), f32),
        scratch_types=[
            pltpu.VMEM((_GRP, KNN), jnp.int32),
            pltpu.VMEM((_GRP, KNN), f32),
            pltpu.VMEM((KNN, d), f32),
            pltpu.VMEM((KNN, d), f32),
            pltpu.VMEM((_GRP, d), f32),
            pltpu.SemaphoreType.DMA,
            pltpu.SemaphoreType.DMA,
        ],
        compiler_params=_sc_compiler_params(),
    )
    return fn(uf, idxg, attn)


def _tail_body(qf_ref, msg_ref, xp_ref,
               wrq_ref, wrk_ref, wd_ref, r_ref, rt_ref, r2_ref, r2t_ref,
               gam_ref, bet_ref, s_ref, sel0_ref, sel1_ref, sel2_ref,
               xc_ref, v0_ref, v1_ref, v2_ref):
    r = r_ref[...]
    rt = rt_ref[...]
    out = qf_ref[...] + 0.5 * msg_ref[...]
    norm = jnp.maximum(jnp.sqrt(_mm(out * out, r)), EPS)
    mean = jnp.mean(norm, axis=1, keepdims=True)
    dev = norm - mean
    nchan = norm.shape[1]
    var = jnp.sum(dev * dev, axis=1, keepdims=True) * (1.0 / (nchan - 1))
    std = jnp.maximum(jnp.sqrt(var), EPS)
    ns = (dev / std) * gam_ref[...] + bet_ref[...]
    fac = jnp.maximum(ns, EPS) / norm
    out = out * _mm(fac, rt)
    n2 = jnp.maximum(jnp.sqrt(_mm(out * out, r)), EPS)
    out = out * _mm(jnp.minimum(CLAMP / n2, 1.0), rt)
    q = _mmd(out, wrq_ref[...])
    kk = _mmd(out, wrk_ref[...])
    dqk = _mm(q * kk, r)
    kn2 = jnp.maximum(_mm(kk * kk, r), EPS * EPS)
    t = jnp.where(dqk >= 0.0, 0.0, dqk / kn2)
    h = q - _mm(t, rt) * kk
    disp = _mmd(h, wd_ref[...])
    dn = jnp.maximum(jnp.sqrt(_mm(disp * disp, r2_ref[...])), EPS)
    scl = jnp.tanh(dn) / dn * 0.1
    disp = disp * _mm(scl, r2t_ref[...])
    xc_ref[...] = _mm(xp_ref[...], s_ref[...]) + disp
    h0 = _mm(h, sel0_ref[...])
    h1v = _mm(h, sel1_ref[...])
    h2v = _mm(h, sel2_ref[...])
    v0_ref[...] = jnp.concatenate([h0, h0], axis=1)
    v1_ref[...] = jnp.concatenate([h1v, h1v], axis=1)
    v2_ref[...] = jnp.concatenate([h2v, h2v], axis=1)


def _tail(qf, msg, xp, wrq3t, wrk3t, wd3t, r_mat, rt_mat, r2, r2t, gam, bet,
          smat, sels, tn=256):
    m, d = qf.shape
    c = d // 3
    grid = (m // tn,)
    f32 = jnp.float32
    return pl.pallas_call(
        _tail_body,
        grid=grid,
        in_specs=[
            pl.BlockSpec((tn, d), lambda i: (i, 0)),
            pl.BlockSpec((tn, d), lambda i: (i, 0)),
            pl.BlockSpec((tn, 8), lambda i: (i, 0)),
            pl.BlockSpec((d, d), lambda i: (0, 0)),
            pl.BlockSpec((d, d), lambda i: (0, 0)),
            pl.BlockSpec((d, 3 * UP), lambda i: (0, 0)),
            pl.BlockSpec((d, c), lambda i: (0, 0)),
            pl.BlockSpec((c, d), lambda i: (0, 0)),
            pl.BlockSpec((3 * UP, UP), lambda i: (0, 0)),
            pl.BlockSpec((UP, 3 * UP), lambda i: (0, 0)),
            pl.BlockSpec((1, c), lambda i: (0, 0)),
            pl.BlockSpec((1, c), lambda i: (0, 0)),
            pl.BlockSpec((8, 3 * UP), lambda i: (0, 0)),
            pl.BlockSpec((d, c), lambda i: (0, 0)),
            pl.BlockSpec((d, c), lambda i: (0, 0)),
            pl.BlockSpec((d, c), lambda i: (0, 0)),
        ],
        out_specs=[
            pl.BlockSpec((tn, 3 * UP), lambda i: (i, 0)),
            pl.BlockSpec((tn, 2 * c), lambda i: (i, 0)),
            pl.BlockSpec((tn, 2 * c), lambda i: (i, 0)),
            pl.BlockSpec((tn, 2 * c), lambda i: (i, 0)),
        ],
        out_shape=[
            jax.ShapeDtypeStruct((m, 3 * UP), f32),
            jax.ShapeDtypeStruct((m, 2 * c), f32),
            jax.ShapeDtypeStruct((m, 2 * c), f32),
            jax.ShapeDtypeStruct((m, 2 * c), f32),
        ],
    )(qf, msg, xp, wrq3t, wrk3t, wd3t, r_mat, rt_mat, r2, r2t, gam, bet, smat,
      *sels)


def kernel(x, v, Wq, Wk, Wu, W1, b1, W2, b2, W3, b3, gamma, beta, Wrq, Wrk,
           Wd):
    bsz, n, c, _ = v.shape
    m = bsz * n
    d = 3 * c
    f32 = jnp.float32
    i3 = jnp.eye(3, dtype=f32)

    vf = v.reshape(m, d)
    x2 = x.reshape(m, 3)
    xp = jnp.concatenate([x2, jnp.zeros((m, 5), f32)], axis=1)
    xpt = jnp.pad(jnp.swapaxes(x, 1, 2), ((0, 0), (0, 5), (0, 0)))

    wq3t = jnp.kron(Wq, i3).T
    wk3t = jnp.kron(Wk, i3).T
    wu3t = jnp.kron(Wu, i3).T
    wrq3t = jnp.kron(Wrq, i3).T
    wrk3t = jnp.kron(Wrk, i3).T
    wd3t = jnp.kron(Wd, i3).T
    r_np = np.kron(np.eye(c, dtype=np.float32), np.ones((3, 1), np.float32))
    r_mat = jnp.asarray(r_np)
    rt_mat = jnp.asarray(r_np.T)
    r2_np = np.kron(np.eye(UP, dtype=np.float32), np.ones((3, 1), np.float32))
    r2 = jnp.asarray(r2_np)
    r2t = jnp.asarray(r2_np.T)
    s_np = np.zeros((8, 3 * UP), np.float32)
    for u in range(UP):
        for dd in range(3):
            s_np[dd, u * 3 + dd] = 1.0
    smat = jnp.asarray(s_np)
    sels = []
    for dd in range(3):
        e_np = np.zeros((3, 1), np.float32)
        e_np[dd, 0] = 1.0
        sels.append(jnp.asarray(np.kron(np.eye(c, dtype=np.float32), e_np)))

    ieye = jnp.eye(KNN, dtype=f32)
    w0t = jnp.tile(W1[0:1, :], (1, KNN))
    k1kn = jnp.kron(ieye, W1[1:2, :])
    k1dot = jnp.kron(ieye, W1[2:3, :])
    k1dist = jnp.kron(ieye, W1[3:4, :])
    b1t = jnp.tile(b1[None, :], (1, KNN))
    k2 = jnp.kron(ieye, W2)
    b2t = jnp.tile(b2[None, :], (1, KNN))
    k3 = jnp.kron(ieye, W3)
    b3t = jnp.tile(b3[None, :], (1, KNN))

    qf, kf, uf, qn, kn = _project(vf, wq3t, wk3t, wu3t, r_mat)
    knt = kn.reshape(bsz, 1, n)
    idxg, ssd16, kn16 = _knn(xp, xpt, knt)
    dot16 = _sc_pass1(qf, kf, idxg)
    attn = _mlp(qn, kn16, dot16, ssd16, w0t, k1kn, k1dot, k1dist, b1t, k2,
                b2t, k3, b3t)
    msg = _sc_pass2(uf, idxg, attn)
    gam = gamma[None, :]
    bet = beta[None, :]
    xc6, v0, v1, v2 = _tail(qf, msg, xp, wrq3t, wrk3t, wd3t, r_mat, rt_mat,
                            r2, r2t, gam, bet, smat, sels)
    x_child = xc6.reshape(bsz, n * UP, 3)
    st = jnp.stack([vv.reshape(bsz, n, UP, c) for vv in (v0, v1, v2)], axis=1)
    v_child = st.reshape(bsz, 3, n * UP, c).transpose(0, 2, 3, 1)
    return x_child, v_child


# speedup vs baseline: 1.1440x; 1.1440x over previous
import dataclasses
import functools

import jax
import jax.numpy as jnp
import numpy as np
from jax import lax
from jax.experimental import pallas as pl
from jax.experimental.pallas import tpu as pltpu
from jax.experimental.pallas import tpu_sc as plsc

EPS = 1e-06
CLAMP = 50.0
KNN = 16
UP = 2
HI = lax.Precision.HIGHEST

_NW = 32
_GRP = 64


def _mm(a, b):
    return jnp.dot(a, b, precision=HI, preferred_element_type=jnp.float32)


def _mmd(a, b):
    return jnp.dot(a, b, precision=lax.Precision.DEFAULT,
                   preferred_element_type=jnp.float32)


def _sc_compiler_params():
    cp = pltpu.CompilerParams()
    if "needs_layout_passes" in pltpu.CompilerParams.__dataclass_fields__:
        cp = dataclasses.replace(cp, needs_layout_passes=False)
    if "use_tc_tiling_on_sc" in pltpu.CompilerParams.__dataclass_fields__:
        cp = dataclasses.replace(cp, use_tc_tiling_on_sc=True)
    return cp


def _proj_body(vf_ref, wq_ref, wk_ref, wu_ref, r_ref,
               qf_ref, kt_ref, uf_ref, qn_ref):
    vb = vf_ref[...]
    r = r_ref[...]
    q = _mmd(vb, wq_ref[...])
    k = _mmd(vb, wk_ref[...])
    u = _mmd(vb, wu_ref[...])
    qf_ref[...] = q
    uf_ref[...] = u
    qn_ref[...] = jnp.mean(jnp.sqrt(_mm(q * q, r)), axis=1, keepdims=True)
    kn = jnp.mean(jnp.sqrt(_mm(k * k, r)), axis=1, keepdims=True)
    tn = k.shape[0]
    pad = jnp.zeros((tn, 127), jnp.float32)
    kt_ref[...] = jnp.concatenate([k, kn, pad], axis=1)


def _project(vf, wq3t, wk3t, wu3t, r_mat, tn=512):
    m, d = vf.shape
    c = d // 3
    grid = (m // tn,)
    f32 = jnp.float32
    return pl.pallas_call(
        _proj_body,
        grid=grid,
        in_specs=[
            pl.BlockSpec((tn, d), lambda i: (i, 0)),
            pl.BlockSpec((d, d), lambda i: (0, 0)),
            pl.BlockSpec((d, d), lambda i: (0, 0)),
            pl.BlockSpec((d, d), lambda i: (0, 0)),
            pl.BlockSpec((d, c), lambda i: (0, 0)),
        ],
        out_specs=[
            pl.BlockSpec((tn, d), lambda i: (i, 0)),
            pl.BlockSpec((tn, d + 128), lambda i: (i, 0)),
            pl.BlockSpec((tn, d), lambda i: (i, 0)),
            pl.BlockSpec((tn, 1), lambda i: (i, 0)),
        ],
        out_shape=[
            jax.ShapeDtypeStruct((m, d), f32),
            jax.ShapeDtypeStruct((m, d + 128), f32),
            jax.ShapeDtypeStruct((m, d), f32),
            jax.ShapeDtypeStruct((m, 1), f32),
        ],
    )(vf, wq3t, wk3t, wu3t, r_mat)


def _knn_body(xr_ref, xct_ref, idx_ref, ssd_ref, *, n, tn):
    b = pl.program_id(0)
    i = pl.program_id(1)
    xr = xr_ref[...]
    xc = xct_ref[0]
    sqr = jnp.sum(xr * xr, axis=1, keepdims=True)
    sqc = jnp.sum(xc * xc, axis=0, keepdims=True)
    dotm = _mmd(xr, xc)
    d2 = jnp.maximum(sqr + sqc - 2.0 * dotm, 0.0)
    col = lax.broadcasted_iota(jnp.int32, (tn, n), 1)
    row = i * tn + lax.broadcasted_iota(jnp.int32, (tn, n), 0)
    big = jnp.int32(0x7FFFFFFF)
    key = ((lax.bitcast_convert_type(d2, jnp.int32) + jnp.int32(0x800))
           & jnp.int32(~0xFFF)) | col
    key = jnp.where(col == row, big, key)
    cols, ssds = [], []
    for _ in range(KNN):
        kmin = jnp.min(key, axis=1, keepdims=True)
        key = jnp.where(key == kmin, big, key)
        cols.append(kmin & jnp.int32(0xFFF))
        ssds.append(lax.bitcast_convert_type(kmin & jnp.int32(~0xFFF),
                                             jnp.float32))
    idx_ref[...] = jnp.concatenate(cols, axis=1) + b * n
    ssd_ref[...] = jnp.concatenate(ssds, axis=1)


def _knn(xp, xpt, tn=256):
    bsz, _, n = xpt.shape
    m = bsz * n
    grid = (bsz, n // tn)
    body = functools.partial(_knn_body, n=n, tn=tn)
    f32 = jnp.float32
    return pl.pallas_call(
        body,
        grid=grid,
        in_specs=[
            pl.BlockSpec((tn, 8), lambda b, i: (b * (n // tn) + i, 0)),
            pl.BlockSpec((1, 8, n), lambda b, i: (b, 0, 0)),
        ],
        out_specs=[
            pl.BlockSpec((tn, KNN), lambda b, i: (b * (n // tn) + i, 0)),
            pl.BlockSpec((tn, KNN), lambda b, i: (b * (n // tn) + i, 0)),
        ],
        out_shape=[
            jax.ShapeDtypeStruct((m, KNN), jnp.int32),
            jax.ShapeDtypeStruct((m, KNN), f32),
        ],
    )(xp, xpt)


def _sc1_body(q_hbm, k_hbm, idx_hbm, dot_hbm, knb_hbm,
              idx_v, q_v, kr0_v, kr1_v, dot_v, knb_v, sem0, sem1):
    c = lax.axis_index("c")
    s = lax.axis_index("s")
    wid = s * 2 + c
    per_w = q_hbm.shape[0] // _NW
    base = wid * per_w
    lanes = lax.iota(jnp.int32, 16)
    last = _GRP - 1

    def compute(p, kr_v):
        qv = [q_v[p, pl.ds(i * 16, 16)] for i in range(24)]
        dotvec = jnp.zeros((16,), jnp.float32)
        knvec = jnp.zeros((16,), jnp.float32)
        for j in range(KNN):
            acc = qv[0] * kr_v[j, pl.ds(0, 16)]
            for i in range(1, 24):
                acc = acc + qv[i] * kr_v[j, pl.ds(i * 16, 16)]
            dsc = jnp.sum(acc)
            knc = kr_v[j, pl.ds(384, 16)][0]
            onehot = lanes == j
            dotvec = jnp.where(onehot, jnp.full((16,), dsc), dotvec)
            knvec = jnp.where(onehot, jnp.full((16,), knc), knvec)
        dot_v[p, :] = dotvec
        knb_v[p, :] = knvec

    @pl.loop(0, per_w // _GRP)
    def _grp(g):
        gb = base + g * _GRP
        pltpu.sync_copy(idx_hbm.at[pl.ds(gb, _GRP)], idx_v)
        pltpu.sync_copy(q_hbm.at[pl.ds(gb, _GRP)], q_v)
        pltpu.async_copy(k_hbm.at[idx_v.at[0]], kr0_v, sem0)

        @pl.loop(0, _GRP // 2)
        def _pt(t):
            p0 = 2 * t
            pltpu.async_copy(k_hbm.at[idx_v.at[p0 + 1]], kr1_v, sem1)
            pltpu.make_async_copy(k_hbm.at[idx_v.at[p0]], kr0_v, sem0).wait()
            compute(p0, kr0_v)
            nxt = jnp.minimum(p0 + 2, last)
            pltpu.async_copy(k_hbm.at[idx_v.at[nxt]], kr0_v, sem0)
            pltpu.make_async_copy(k_hbm.at[idx_v.at[p0 + 1]], kr1_v,
                                  sem1).wait()
            compute(p0 + 1, kr1_v)

        pltpu.make_async_copy(k_hbm.at[idx_v.at[last]], kr0_v, sem0).wait()
        pltpu.sync_copy(dot_v, dot_hbm.at[pl.ds(gb, _GRP)])
        pltpu.sync_copy(knb_v, knb_hbm.at[pl.ds(gb, _GRP)])


def _sc_pass1(qf, ktab, idxg):
    m, d = qf.shape
    dk = ktab.shape[1]
    f32 = jnp.float32
    mesh = plsc.VectorSubcoreMesh(core_axis_name="c", subcore_axis_name="s")
    fn = pl.kernel(
        _sc1_body,
        mesh=mesh,
        out_type=[
            jax.ShapeDtypeStruct((m, KNN), f32),
            jax.ShapeDtypeStruct((m, KNN), f32),
        ],
        scratch_types=[
            pltpu.VMEM((_GRP, KNN), jnp.int32),
            pltpu.VMEM((_GRP, d), f32),
            pltpu.VMEM((KNN, dk), f32),
            pltpu.VMEM((KNN, dk), f32),
            pltpu.VMEM((_GRP, KNN), f32),
            pltpu.VMEM((_GRP, KNN), f32),
            pltpu.SemaphoreType.DMA,
            pltpu.SemaphoreType.DMA,
        ],
        compiler_params=_sc_compiler_params(),
    )
    return fn(qf, ktab, idxg)


def _mlp_body(qn_ref, knb_ref, dot_ref, ssd_ref,
              w0_ref, k1kn_ref, k1dot_ref, k1dist_ref, b1_ref,
              k2_ref, b2_ref, k3_ref, b3_ref, attn_ref):
    qn = qn_ref[...]
    knb = knb_ref[...]
    dot = dot_ref[...] * (1.0 / 128.0)
    dist = jnp.sqrt(ssd_ref[...])

    def silu(t):
        return t * (1.0 / (1.0 + jnp.exp(-t)))

    h1 = (_mmd(qn, w0_ref[...]) + _mmd(knb, k1kn_ref[...])
          + _mmd(dot, k1dot_ref[...]) + _mmd(dist, k1dist_ref[...])
          + b1_ref[...])
    h1 = silu(h1)
    h2 = silu(_mmd(h1, k2_ref[...]) + b2_ref[...])
    logits = _mmd(h2, k3_ref[...]) + b3_ref[...]
    logits = jnp.clip(logits, -10.0, 10.0)
    mx = jnp.max(logits, axis=1, keepdims=True)
    e = jnp.exp(logits - mx)
    attn_ref[...] = e / jnp.sum(e, axis=1, keepdims=True)


def _mlp(qn, knb, dot16, ssd16, w0t, k1kn, k1dot, k1dist, b1t, k2, b2t, k3,
         b3t, tn=512):
    m = qn.shape[0]
    h16 = k2.shape[0]
    grid = (m // tn,)
    return pl.pallas_call(
        _mlp_body,
        grid=grid,
        in_specs=[
            pl.BlockSpec((tn, 1), lambda i: (i, 0)),
            pl.BlockSpec((tn, KNN), lambda i: (i, 0)),
            pl.BlockSpec((tn, KNN), lambda i: (i, 0)),
            pl.BlockSpec((tn, KNN), lambda i: (i, 0)),
            pl.BlockSpec((1, h16), lambda i: (0, 0)),
            pl.BlockSpec((KNN, h16), lambda i: (0, 0)),
            pl.BlockSpec((KNN, h16), lambda i: (0, 0)),
            pl.BlockSpec((KNN, h16), lambda i: (0, 0)),
            pl.BlockSpec((1, h16), lambda i: (0, 0)),
            pl.BlockSpec((h16, h16), lambda i: (0, 0)),
            pl.BlockSpec((1, h16), lambda i: (0, 0)),
            pl.BlockSpec((h16, KNN), lambda i: (0, 0)),
            pl.BlockSpec((1, KNN), lambda i: (0, 0)),
        ],
        out_specs=pl.BlockSpec((tn, KNN), lambda i: (i, 0)),
        out_shape=jax.ShapeDtypeStruct((m, KNN), jnp.float32),
    )(qn, knb, dot16, ssd16, w0t, k1kn, k1dot, k1dist, b1t, k2, b2t, k3, b3t)


def _sc2_body(u_hbm, idx_hbm, attn_hbm, msg_hbm,
              idx_v, attn_v, ur0_v, ur1_v, msg_v, sem0, sem1):
    c = lax.axis_index("c")
    s = lax.axis_index("s")
    wid = s * 2 + c
    per_w = msg_hbm.shape[0] // _NW
    base = wid * per_w
    last = _GRP - 1

    def compute(p, ur_v):
        arow = attn_v[p, :]
        accs = [jnp.zeros((16,), jnp.float32) for _ in range(24)]
        for j in range(KNN):
            wv = jnp.full((16,), arow[j])
            for i in range(24):
                accs[i] = accs[i] + wv * ur_v[j, pl.ds(i * 16, 16)]
        for i in range(24):
            msg_v[p, pl.ds(i * 16, 16)] = accs[i]

    @pl.loop(0, per_w // _GRP)
    def _grp(g):
        gb = base + g * _GRP
        pltpu.sync_copy(idx_hbm.at[pl.ds(gb, _GRP)], idx_v)
        pltpu.sync_copy(attn_hbm.at[pl.ds(gb, _GRP)], attn_v)
        pltpu.async_copy(u_hbm.at[idx_v.at[0]], ur0_v, sem0)

        @pl.loop(0, _GRP // 2)
        def _pt(t):
            p0 = 2 * t
            pltpu.async_copy(u_hbm.at[idx_v.at[p0 + 1]], ur1_v, sem1)
            pltpu.make_async_copy(u_hbm.at[idx_v.at[p0]], ur0_v, sem0).wait()
            compute(p0, ur0_v)
            nxt = jnp.minimum(p0 + 2, last)
            pltpu.async_copy(u_hbm.at[idx_v.at[nxt]], ur0_v, sem0)
            pltpu.make_async_copy(u_hbm.at[idx_v.at[p0 + 1]], ur1_v,
                                  sem1).wait()
            compute(p0 + 1, ur1_v)

        pltpu.make_async_copy(u_hbm.at[idx_v.at[last]], ur0_v, sem0).wait()
        pltpu.sync_copy(msg_v, msg_hbm.at[pl.ds(gb, _GRP)])


def _sc_pass2(uf, idxg, attn):
    m, d = uf.shape
    f32 = jnp.float32
    mesh = plsc.VectorSubcoreMesh(core_axis_name="c", subcore_axis_name="s")
    fn = pl.kernel(
        _sc2_body,
        mesh=mesh,
        out_type=jax.ShapeDtypeStruct((m, d), f32),
        scratch_types=[
            pltpu.VMEM((_GRP, KNN), jnp.int32),
            pltpu.VMEM((_GRP, KNN), f32),
            pltpu.VMEM((KNN, d), f32),
            pltpu.VMEM((KNN, d), f32),
            pltpu.VMEM((_GRP, d), f32),
            pltpu.SemaphoreType.DMA,
            pltpu.SemaphoreType.DMA,
        ],
        compiler_params=_sc_compiler_params(),
    )
    return fn(uf, idxg, attn)


def _tail_body(qf_ref, msg_ref, xp_ref,
               wrq_ref, wrk_ref, wd_ref, r_ref, rt_ref, r2_ref, r2t_ref,
               gam_ref, bet_ref, s_ref, sel0_ref, sel1_ref, sel2_ref,
               xc_ref, v0_ref, v1_ref, v2_ref):
    r = r_ref[...]
    rt = rt_ref[...]
    out = qf_ref[...] + 0.5 * msg_ref[...]
    norm = jnp.maximum(jnp.sqrt(_mm(out * out, r)), EPS)
    mean = jnp.mean(norm, axis=1, keepdims=True)
    dev = norm - mean
    nchan = norm.shape[1]
    var = jnp.sum(dev * dev, axis=1, keepdims=True) * (1.0 / (nchan - 1))
    std = jnp.maximum(jnp.sqrt(var), EPS)
    ns = (dev / std) * gam_ref[...] + bet_ref[...]
    fac = jnp.maximum(ns, EPS) / norm
    out = out * _mm(fac, rt)
    n2 = jnp.maximum(jnp.sqrt(_mm(out * out, r)), EPS)
    out = out * _mm(jnp.minimum(CLAMP / n2, 1.0), rt)
    q = _mmd(out, wrq_ref[...])
    kk = _mmd(out, wrk_ref[...])
    dqk = _mm(q * kk, r)
    kn2 = jnp.maximum(_mm(kk * kk, r), EPS * EPS)
    t = jnp.where(dqk >= 0.0, 0.0, dqk / kn2)
    h = q - _mm(t, rt) * kk
    disp = _mmd(h, wd_ref[...])
    dn = jnp.maximum(jnp.sqrt(_mm(disp * disp, r2_ref[...])), EPS)
    scl = jnp.tanh(dn) / dn * 0.1
    disp = disp * _mm(scl, r2t_ref[...])
    xc_ref[...] = _mm(xp_ref[...], s_ref[...]) + disp
    h0 = _mm(h, sel0_ref[...])
    h1v = _mm(h, sel1_ref[...])
    h2v = _mm(h, sel2_ref[...])
    v0_ref[...] = jnp.concatenate([h0, h0], axis=1)
    v1_ref[...] = jnp.concatenate([h1v, h1v], axis=1)
    v2_ref[...] = jnp.concatenate([h2v, h2v], axis=1)


def _tail(qf, msg, xp, wrq3t, wrk3t, wd3t, r_mat, rt_mat, r2, r2t, gam, bet,
          smat, sels, tn=256):
    m, d = qf.shape
    c = d // 3
    grid = (m // tn,)
    f32 = jnp.float32
    return pl.pallas_call(
        _tail_body,
        grid=grid,
        in_specs=[
            pl.BlockSpec((tn, d), lambda i: (i, 0)),
            pl.BlockSpec((tn, d), lambda i: (i, 0)),
            pl.BlockSpec((tn, 8), lambda i: (i, 0)),
            pl.BlockSpec((d, d), lambda i: (0, 0)),
            pl.BlockSpec((d, d), lambda i: (0, 0)),
            pl.BlockSpec((d, 3 * UP), lambda i: (0, 0)),
            pl.BlockSpec((d, c), lambda i: (0, 0)),
            pl.BlockSpec((c, d), lambda i: (0, 0)),
            pl.BlockSpec((3 * UP, UP), lambda i: (0, 0)),
            pl.BlockSpec((UP, 3 * UP), lambda i: (0, 0)),
            pl.BlockSpec((1, c), lambda i: (0, 0)),
            pl.BlockSpec((1, c), lambda i: (0, 0)),
            pl.BlockSpec((8, 3 * UP), lambda i: (0, 0)),
            pl.BlockSpec((d, c), lambda i: (0, 0)),
            pl.BlockSpec((d, c), lambda i: (0, 0)),
            pl.BlockSpec((d, c), lambda i: (0, 0)),
        ],
        out_specs=[
            pl.BlockSpec((tn, 3 * UP), lambda i: (i, 0)),
            pl.BlockSpec((tn, 2 * c), lambda i: (i, 0)),
            pl.BlockSpec((tn, 2 * c), lambda i: (i, 0)),
            pl.BlockSpec((tn, 2 * c), lambda i: (i, 0)),
        ],
        out_shape=[
            jax.ShapeDtypeStruct((m, 3 * UP), f32),
            jax.ShapeDtypeStruct((m, 2 * c), f32),
            jax.ShapeDtypeStruct((m, 2 * c), f32),
            jax.ShapeDtypeStruct((m, 2 * c), f32),
        ],
    )(qf, msg, xp, wrq3t, wrk3t, wd3t, r_mat, rt_mat, r2, r2t, gam, bet, smat,
      *sels)


def kernel(x, v, Wq, Wk, Wu, W1, b1, W2, b2, W3, b3, gamma, beta, Wrq, Wrk,
           Wd):
    bsz, n, c, _ = v.shape
    m = bsz * n
    d = 3 * c
    f32 = jnp.float32
    i3 = jnp.eye(3, dtype=f32)

    vf = v.reshape(m, d)
    x2 = x.reshape(m, 3)
    xp = jnp.concatenate([x2, jnp.zeros((m, 5), f32)], axis=1)
    xpt = jnp.pad(jnp.swapaxes(x, 1, 2), ((0, 0), (0, 5), (0, 0)))

    wq3t = jnp.kron(Wq, i3).T
    wk3t = jnp.kron(Wk, i3).T
    wu3t = jnp.kron(Wu, i3).T
    wrq3t = jnp.kron(Wrq, i3).T
    wrk3t = jnp.kron(Wrk, i3).T
    wd3t = jnp.kron(Wd, i3).T
    r_np = np.kron(np.eye(c, dtype=np.float32), np.ones((3, 1), np.float32))
    r_mat = jnp.asarray(r_np)
    rt_mat = jnp.asarray(r_np.T)
    r2_np = np.kron(np.eye(UP, dtype=np.float32), np.ones((3, 1), np.float32))
    r2 = jnp.asarray(r2_np)
    r2t = jnp.asarray(r2_np.T)
    s_np = np.zeros((8, 3 * UP), np.float32)
    for u in range(UP):
        for dd in range(3):
            s_np[dd, u * 3 + dd] = 1.0
    smat = jnp.asarray(s_np)
    sels = []
    for dd in range(3):
        e_np = np.zeros((3, 1), np.float32)
        e_np[dd, 0] = 1.0
        sels.append(jnp.asarray(np.kron(np.eye(c, dtype=np.float32), e_np)))

    ieye = jnp.eye(KNN, dtype=f32)
    w0t = jnp.tile(W1[0:1, :], (1, KNN))
    k1kn = jnp.kron(ieye, W1[1:2, :])
    k1dot = jnp.kron(ieye, W1[2:3, :])
    k1dist = jnp.kron(ieye, W1[3:4, :])
    b1t = jnp.tile(b1[None, :], (1, KNN))
    k2 = jnp.kron(ieye, W2)
    b2t = jnp.tile(b2[None, :], (1, KNN))
    k3 = jnp.kron(ieye, W3)
    b3t = jnp.tile(b3[None, :], (1, KNN))

    qf, ktab, uf, qn = _project(vf, wq3t, wk3t, wu3t, r_mat)
    idxg, ssd16 = _knn(xp, xpt)
    dot16, kn16 = _sc_pass1(qf, ktab, idxg)
    attn = _mlp(qn, kn16, dot16, ssd16, w0t, k1kn, k1dot, k1dist, b1t, k2,
                b2t, k3, b3t)
    msg = _sc_pass2(uf, idxg, attn)
    gam = gamma[None, :]
    bet = beta[None, :]
    xc6, v0, v1, v2 = _tail(qf, msg, xp, wrq3t, wrk3t, wd3t, r_mat, rt_mat,
                            r2, r2t, gam, bet, smat, sels)
    x_child = xc6.reshape(bsz, n * UP, 3)
    st = jnp.stack([vv.reshape(bsz, n, UP, c) for vv in (v0, v1, v2)], axis=1)
    v_child = st.reshape(bsz, 3, n * UP, c).transpose(0, 2, 3, 1)
    return x_child, v_child


# speedup vs baseline: 1.3757x; 1.2025x over previous
import dataclasses
import functools

import jax
import jax.numpy as jnp
import numpy as np
from jax import lax
from jax.experimental import pallas as pl
from jax.experimental.pallas import tpu as pltpu
from jax.experimental.pallas import tpu_sc as plsc

EPS = 1e-06
CLAMP = 50.0
KNN = 16
UP = 2
HI = lax.Precision.HIGHEST

_NW = 32
_GRP = 64


def _mm(a, b):
    return jnp.dot(a, b, precision=HI, preferred_element_type=jnp.float32)


def _mmd(a, b):
    return jnp.dot(a, b, precision=lax.Precision.DEFAULT,
                   preferred_element_type=jnp.float32)


def _sc_compiler_params():
    cp = pltpu.CompilerParams()
    if "needs_layout_passes" in pltpu.CompilerParams.__dataclass_fields__:
        cp = dataclasses.replace(cp, needs_layout_passes=False)
    if "use_tc_tiling_on_sc" in pltpu.CompilerParams.__dataclass_fields__:
        cp = dataclasses.replace(cp, use_tc_tiling_on_sc=True)
    return cp


def _proj_body(vf_ref, wq_ref, wk_ref, wu_ref, r_ref,
               qf_ref, kt_ref, uf_ref, qn_ref):
    vb = vf_ref[...]
    r = r_ref[...]
    q = _mmd(vb, wq_ref[...])
    k = _mmd(vb, wk_ref[...])
    u = _mmd(vb, wu_ref[...])
    qf_ref[...] = q
    uf_ref[...] = u
    qn_ref[...] = jnp.mean(jnp.sqrt(_mm(q * q, r)), axis=1, keepdims=True)
    kn = jnp.mean(jnp.sqrt(_mm(k * k, r)), axis=1, keepdims=True)
    tn = k.shape[0]
    pad = jnp.zeros((tn, 127), jnp.float32)
    kt_ref[...] = jnp.concatenate([k, kn, pad], axis=1)


def _project(vf, wq3t, wk3t, wu3t, r_mat, tn=512):
    m, d = vf.shape
    c = d // 3
    grid = (m // tn,)
    f32 = jnp.float32
    return pl.pallas_call(
        _proj_body,
        grid=grid,
        in_specs=[
            pl.BlockSpec((tn, d), lambda i: (i, 0)),
            pl.BlockSpec((d, d), lambda i: (0, 0)),
            pl.BlockSpec((d, d), lambda i: (0, 0)),
            pl.BlockSpec((d, d), lambda i: (0, 0)),
            pl.BlockSpec((d, c), lambda i: (0, 0)),
        ],
        out_specs=[
            pl.BlockSpec((tn, d), lambda i: (i, 0)),
            pl.BlockSpec((tn, d + 128), lambda i: (i, 0)),
            pl.BlockSpec((tn, d), lambda i: (i, 0)),
            pl.BlockSpec((tn, 1), lambda i: (i, 0)),
        ],
        out_shape=[
            jax.ShapeDtypeStruct((m, d), f32),
            jax.ShapeDtypeStruct((m, d + 128), f32),
            jax.ShapeDtypeStruct((m, d), f32),
            jax.ShapeDtypeStruct((m, 1), f32),
        ],
    )(vf, wq3t, wk3t, wu3t, r_mat)


def _knn_body(xr_ref, xct_ref, idx_ref, ssd_ref, *, n, tn):
    i = pl.program_id(0)
    xr = xr_ref[...]
    xc = xct_ref[...]
    sqr = jnp.sum(xr * xr, axis=1, keepdims=True)
    sqc = jnp.sum(xc * xc, axis=0, keepdims=True)
    dotm = _mmd(xr, xc)
    d2 = jnp.maximum(sqr + sqc - 2.0 * dotm, 0.0)
    col = lax.broadcasted_iota(jnp.int32, (tn, n), 1)
    row = i * tn + lax.broadcasted_iota(jnp.int32, (tn, n), 0)
    big = jnp.int32(0x7FFFFFFF)
    key = ((lax.bitcast_convert_type(d2, jnp.int32) + jnp.int32(0x800))
           & jnp.int32(~0xFFF)) | col
    key = jnp.where(col == row, big, key)
    cols, ssds = [], []
    for _ in range(KNN):
        kmin = jnp.min(key, axis=1, keepdims=True)
        key = jnp.where(key == kmin, big, key)
        cols.append(kmin & jnp.int32(0xFFF))
        ssds.append(lax.bitcast_convert_type(kmin & jnp.int32(~0xFFF),
                                             jnp.float32))
    idx_ref[...] = jnp.concatenate(cols, axis=1)
    ssd_ref[...] = jnp.concatenate(ssds, axis=1)


def _knn(xp, xpt, tn=256):
    _, n = xpt.shape
    grid = (n // tn,)
    body = functools.partial(_knn_body, n=n, tn=tn)
    f32 = jnp.float32
    return pl.pallas_call(
        body,
        grid=grid,
        in_specs=[
            pl.BlockSpec((tn, 8), lambda i: (i, 0)),
            pl.BlockSpec((8, n), lambda i: (0, 0)),
        ],
        out_specs=[
            pl.BlockSpec((tn, KNN), lambda i: (i, 0)),
            pl.BlockSpec((tn, KNN), lambda i: (i, 0)),
        ],
        out_shape=[
            jax.ShapeDtypeStruct((n, KNN), jnp.int32),
            jax.ShapeDtypeStruct((n, KNN), f32),
        ],
    )(xp, xpt)


def _sc1_body(q_hbm, k_hbm, idx_hbm, dot_hbm, knb_hbm,
              idx_v, q_v, kr0_v, kr1_v, dot_v, knb_v, sem0, sem1):
    c = lax.axis_index("c")
    s = lax.axis_index("s")
    wid = s * 2 + c
    per_w = q_hbm.shape[0] // _NW
    base = wid * per_w
    lanes = lax.iota(jnp.int32, 16)
    last = _GRP - 1

    def compute(p, kr_v):
        qv = [q_v[p, pl.ds(i * 16, 16)] for i in range(24)]
        dotvec = jnp.zeros((16,), jnp.float32)
        knvec = jnp.zeros((16,), jnp.float32)
        for j in range(KNN):
            acc = qv[0] * kr_v[j, pl.ds(0, 16)]
            for i in range(1, 24):
                acc = acc + qv[i] * kr_v[j, pl.ds(i * 16, 16)]
            dsc = jnp.sum(acc)
            knc = kr_v[j, pl.ds(384, 16)][0]
            onehot = lanes == j
            dotvec = jnp.where(onehot, jnp.full((16,), dsc), dotvec)
            knvec = jnp.where(onehot, jnp.full((16,), knc), knvec)
        dot_v[p, :] = dotvec
        knb_v[p, :] = knvec

    @pl.loop(0, per_w // _GRP)
    def _grp(g):
        gb = base + g * _GRP
        pltpu.sync_copy(idx_hbm.at[pl.ds(gb, _GRP)], idx_v)
        pltpu.sync_copy(q_hbm.at[pl.ds(gb, _GRP)], q_v)
        pltpu.async_copy(k_hbm.at[idx_v.at[0]], kr0_v, sem0)

        @pl.loop(0, _GRP // 2)
        def _pt(t):
            p0 = 2 * t
            pltpu.async_copy(k_hbm.at[idx_v.at[p0 + 1]], kr1_v, sem1)
            pltpu.make_async_copy(k_hbm.at[idx_v.at[p0]], kr0_v, sem0).wait()
            compute(p0, kr0_v)
            nxt = jnp.minimum(p0 + 2, last)
            pltpu.async_copy(k_hbm.at[idx_v.at[nxt]], kr0_v, sem0)
            pltpu.make_async_copy(k_hbm.at[idx_v.at[p0 + 1]], kr1_v,
                                  sem1).wait()
            compute(p0 + 1, kr1_v)

        pltpu.make_async_copy(k_hbm.at[idx_v.at[last]], kr0_v, sem0).wait()
        pltpu.sync_copy(dot_v, dot_hbm.at[pl.ds(gb, _GRP)])
        pltpu.sync_copy(knb_v, knb_hbm.at[pl.ds(gb, _GRP)])


def _sc_pass1(qf, ktab, idxg):
    m, d = qf.shape
    dk = ktab.shape[1]
    f32 = jnp.float32
    mesh = plsc.VectorSubcoreMesh(core_axis_name="c", subcore_axis_name="s")
    fn = pl.kernel(
        _sc1_body,
        mesh=mesh,
        out_type=[
            jax.ShapeDtypeStruct((m, KNN), f32),
            jax.ShapeDtypeStruct((m, KNN), f32),
        ],
        scratch_types=[
            pltpu.VMEM((_GRP, KNN), jnp.int32),
            pltpu.VMEM((_GRP, d), f32),
            pltpu.VMEM((KNN, dk), f32),
            pltpu.VMEM((KNN, dk), f32),
            pltpu.VMEM((_GRP, KNN), f32),
            pltpu.VMEM((_GRP, KNN), f32),
            pltpu.SemaphoreType.DMA,
            pltpu.SemaphoreType.DMA,
        ],
        compiler_params=_sc_compiler_params(),
    )
    return fn(qf, ktab, idxg)


def _mlp_body(qn_ref, knb_ref, dot_ref, ssd_ref,
              w0_ref, k1kn_ref, k1dot_ref, k1dist_ref, b1_ref,
              k2_ref, b2_ref, k3_ref, b3_ref, attn_ref):
    qn = qn_ref[...]
    knb = knb_ref[...]
    dot = dot_ref[...] * (1.0 / 128.0)
    dist = jnp.sqrt(ssd_ref[...])

    def silu(t):
        return t * (1.0 / (1.0 + jnp.exp(-t)))

    h1 = (_mmd(qn, w0_ref[...]) + _mmd(knb, k1kn_ref[...])
          + _mmd(dot, k1dot_ref[...]) + _mmd(dist, k1dist_ref[...])
          + b1_ref[...])
    h1 = silu(h1)
    h2 = silu(_mmd(h1, k2_ref[...]) + b2_ref[...])
    logits = _mmd(h2, k3_ref[...]) + b3_ref[...]
    logits = jnp.clip(logits, -10.0, 10.0)
    mx = jnp.max(logits, axis=1, keepdims=True)
    e = jnp.exp(logits - mx)
    attn_ref[...] = e / jnp.sum(e, axis=1, keepdims=True)


def _mlp(qn, knb, dot16, ssd16, w0t, k1kn, k1dot, k1dist, b1t, k2, b2t, k3,
         b3t, tn=512):
    m = qn.shape[0]
    h16 = k2.shape[0]
    grid = (m // tn,)
    return pl.pallas_call(
        _mlp_body,
        grid=grid,
        in_specs=[
            pl.BlockSpec((tn, 1), lambda i: (i, 0)),
            pl.BlockSpec((tn, KNN), lambda i: (i, 0)),
            pl.BlockSpec((tn, KNN), lambda i: (i, 0)),
            pl.BlockSpec((tn, KNN), lambda i: (i, 0)),
            pl.BlockSpec((1, h16), lambda i: (0, 0)),
            pl.BlockSpec((KNN, h16), lambda i: (0, 0)),
            pl.BlockSpec((KNN, h16), lambda i: (0, 0)),
            pl.BlockSpec((KNN, h16), lambda i: (0, 0)),
            pl.BlockSpec((1, h16), lambda i: (0, 0)),
            pl.BlockSpec((h16, h16), lambda i: (0, 0)),
            pl.BlockSpec((1, h16), lambda i: (0, 0)),
            pl.BlockSpec((h16, KNN), lambda i: (0, 0)),
            pl.BlockSpec((1, KNN), lambda i: (0, 0)),
        ],
        out_specs=pl.BlockSpec((tn, KNN), lambda i: (i, 0)),
        out_shape=jax.ShapeDtypeStruct((m, KNN), jnp.float32),
    )(qn, knb, dot16, ssd16, w0t, k1kn, k1dot, k1dist, b1t, k2, b2t, k3, b3t)


def _sc2_body(u_hbm, idx_hbm, attn_hbm, msg_hbm,
              idx_v, attn_v, ur0_v, ur1_v, msg_v, sem0, sem1):
    c = lax.axis_index("c")
    s = lax.axis_index("s")
    wid = s * 2 + c
    per_w = msg_hbm.shape[0] // _NW
    base = wid * per_w
    last = _GRP - 1

    def compute(p, ur_v):
        arow = attn_v[p, :]
        accs = [jnp.zeros((16,), jnp.float32) for _ in range(24)]
        for j in range(KNN):
            wv = jnp.full((16,), arow[j])
            for i in range(24):
                accs[i] = accs[i] + wv * ur_v[j, pl.ds(i * 16, 16)]
        for i in range(24):
            msg_v[p, pl.ds(i * 16, 16)] = accs[i]

    @pl.loop(0, per_w // _GRP)
    def _grp(g):
        gb = base + g * _GRP
        pltpu.sync_copy(idx_hbm.at[pl.ds(gb, _GRP)], idx_v)
        pltpu.sync_copy(attn_hbm.at[pl.ds(gb, _GRP)], attn_v)
        pltpu.async_copy(u_hbm.at[idx_v.at[0]], ur0_v, sem0)

        @pl.loop(0, _GRP // 2)
        def _pt(t):
            p0 = 2 * t
            pltpu.async_copy(u_hbm.at[idx_v.at[p0 + 1]], ur1_v, sem1)
            pltpu.make_async_copy(u_hbm.at[idx_v.at[p0]], ur0_v, sem0).wait()
            compute(p0, ur0_v)
            nxt = jnp.minimum(p0 + 2, last)
            pltpu.async_copy(u_hbm.at[idx_v.at[nxt]], ur0_v, sem0)
            pltpu.make_async_copy(u_hbm.at[idx_v.at[p0 + 1]], ur1_v,
                                  sem1).wait()
            compute(p0 + 1, ur1_v)

        pltpu.make_async_copy(u_hbm.at[idx_v.at[last]], ur0_v, sem0).wait()
        pltpu.sync_copy(msg_v, msg_hbm.at[pl.ds(gb, _GRP)])


def _sc_pass2(uf, idxg, attn):
    m, d = uf.shape
    f32 = jnp.float32
    mesh = plsc.VectorSubcoreMesh(core_axis_name="c", subcore_axis_name="s")
    fn = pl.kernel(
        _sc2_body,
        mesh=mesh,
        out_type=jax.ShapeDtypeStruct((m, d), f32),
        scratch_types=[
            pltpu.VMEM((_GRP, KNN), jnp.int32),
            pltpu.VMEM((_GRP, KNN), f32),
            pltpu.VMEM((KNN, d), f32),
            pltpu.VMEM((KNN, d), f32),
            pltpu.VMEM((_GRP, d), f32),
            pltpu.SemaphoreType.DMA,
            pltpu.SemaphoreType.DMA,
        ],
        compiler_params=_sc_compiler_params(),
    )
    return fn(uf, idxg, attn)


def _tail_body(qf_ref, msg_ref, xp_ref,
               wrq_ref, wrk_ref, wd_ref, r_ref, rt_ref, r2_ref, r2t_ref,
               gam_ref, bet_ref, s_ref, sel0_ref, sel1_ref, sel2_ref,
               xc_ref, v0_ref, v1_ref, v2_ref):
    r = r_ref[...]
    rt = rt_ref[...]
    out = qf_ref[...] + 0.5 * msg_ref[...]
    norm = jnp.maximum(jnp.sqrt(_mm(out * out, r)), EPS)
    mean = jnp.mean(norm, axis=1, keepdims=True)
    dev = norm - mean
    nchan = norm.shape[1]
    var = jnp.sum(dev * dev, axis=1, keepdims=True) * (1.0 / (nchan - 1))
    std = jnp.maximum(jnp.sqrt(var), EPS)
    ns = (dev / std) * gam_ref[...] + bet_ref[...]
    fac = jnp.maximum(ns, EPS) / norm
    out = out * _mm(fac, rt)
    n2 = jnp.maximum(jnp.sqrt(_mm(out * out, r)), EPS)
    out = out * _mm(jnp.minimum(CLAMP / n2, 1.0), rt)
    q = _mmd(out, wrq_ref[...])
    kk = _mmd(out, wrk_ref[...])
    dqk = _mm(q * kk, r)
    kn2 = jnp.maximum(_mm(kk * kk, r), EPS * EPS)
    t = jnp.where(dqk >= 0.0, 0.0, dqk / kn2)
    h = q - _mm(t, rt) * kk
    disp = _mmd(h, wd_ref[...])
    dn = jnp.maximum(jnp.sqrt(_mm(disp * disp, r2_ref[...])), EPS)
    scl = jnp.tanh(dn) / dn * 0.1
    disp = disp * _mm(scl, r2t_ref[...])
    xc_ref[...] = _mm(xp_ref[...], s_ref[...]) + disp
    h0 = _mm(h, sel0_ref[...])
    h1v = _mm(h, sel1_ref[...])
    h2v = _mm(h, sel2_ref[...])
    v0_ref[...] = jnp.concatenate([h0, h0], axis=1)
    v1_ref[...] = jnp.concatenate([h1v, h1v], axis=1)
    v2_ref[...] = jnp.concatenate([h2v, h2v], axis=1)


def _tail(qf, msg, xp, wrq3t, wrk3t, wd3t, r_mat, rt_mat, r2, r2t, gam, bet,
          smat, sels, tn=256):
    m, d = qf.shape
    c = d // 3
    grid = (m // tn,)
    f32 = jnp.float32
    return pl.pallas_call(
        _tail_body,
        grid=grid,
        in_specs=[
            pl.BlockSpec((tn, d), lambda i: (i, 0)),
            pl.BlockSpec((tn, d), lambda i: (i, 0)),
            pl.BlockSpec((tn, 8), lambda i: (i, 0)),
            pl.BlockSpec((d, d), lambda i: (0, 0)),
            pl.BlockSpec((d, d), lambda i: (0, 0)),
            pl.BlockSpec((d, 3 * UP), lambda i: (0, 0)),
            pl.BlockSpec((d, c), lambda i: (0, 0)),
            pl.BlockSpec((c, d), lambda i: (0, 0)),
            pl.BlockSpec((3 * UP, UP), lambda i: (0, 0)),
            pl.BlockSpec((UP, 3 * UP), lambda i: (0, 0)),
            pl.BlockSpec((1, c), lambda i: (0, 0)),
            pl.BlockSpec((1, c), lambda i: (0, 0)),
            pl.BlockSpec((8, 3 * UP), lambda i: (0, 0)),
            pl.BlockSpec((d, c), lambda i: (0, 0)),
            pl.BlockSpec((d, c), lambda i: (0, 0)),
            pl.BlockSpec((d, c), lambda i: (0, 0)),
        ],
        out_specs=[
            pl.BlockSpec((tn, 3 * UP), lambda i: (i, 0)),
            pl.BlockSpec((tn, 2 * c), lambda i: (i, 0)),
            pl.BlockSpec((tn, 2 * c), lambda i: (i, 0)),
            pl.BlockSpec((tn, 2 * c), lambda i: (i, 0)),
        ],
        out_shape=[
            jax.ShapeDtypeStruct((m, 3 * UP), f32),
            jax.ShapeDtypeStruct((m, 2 * c), f32),
            jax.ShapeDtypeStruct((m, 2 * c), f32),
            jax.ShapeDtypeStruct((m, 2 * c), f32),
        ],
    )(qf, msg, xp, wrq3t, wrk3t, wd3t, r_mat, rt_mat, r2, r2t, gam, bet, smat,
      *sels)


def kernel(x, v, Wq, Wk, Wu, W1, b1, W2, b2, W3, b3, gamma, beta, Wrq, Wrk,
           Wd):
    bsz, n, c, _ = v.shape
    m = bsz * n
    d = 3 * c
    f32 = jnp.float32
    i3 = jnp.eye(3, dtype=f32)

    vf = v.reshape(bsz, n, d)
    xp = jnp.concatenate([x, jnp.zeros((bsz, n, 5), f32)], axis=2)
    xpt = jnp.pad(jnp.swapaxes(x, 1, 2), ((0, 0), (0, 5), (0, 0)))

    wq3t = jnp.kron(Wq, i3).T
    wk3t = jnp.kron(Wk, i3).T
    wu3t = jnp.kron(Wu, i3).T
    wrq3t = jnp.kron(Wrq, i3).T
    wrk3t = jnp.kron(Wrk, i3).T
    wd3t = jnp.kron(Wd, i3).T
    r_np = np.kron(np.eye(c, dtype=np.float32), np.ones((3, 1), np.float32))
    r_mat = jnp.asarray(r_np)
    rt_mat = jnp.asarray(r_np.T)
    r2_np = np.kron(np.eye(UP, dtype=np.float32), np.ones((3, 1), np.float32))
    r2 = jnp.asarray(r2_np)
    r2t = jnp.asarray(r2_np.T)
    s_np = np.zeros((8, 3 * UP), np.float32)
    for u in range(UP):
        for dd in range(3):
            s_np[dd, u * 3 + dd] = 1.0
    smat = jnp.asarray(s_np)
    sels = []
    for dd in range(3):
        e_np = np.zeros((3, 1), np.float32)
        e_np[dd, 0] = 1.0
        sels.append(jnp.asarray(np.kron(np.eye(c, dtype=np.float32), e_np)))

    ieye = jnp.eye(KNN, dtype=f32)
    w0t = jnp.tile(W1[0:1, :], (1, KNN))
    k1kn = jnp.kron(ieye, W1[1:2, :])
    k1dot = jnp.kron(ieye, W1[2:3, :])
    k1dist = jnp.kron(ieye, W1[3:4, :])
    b1t = jnp.tile(b1[None, :], (1, KNN))
    k2 = jnp.kron(ieye, W2)
    b2t = jnp.tile(b2[None, :], (1, KNN))
    k3 = jnp.kron(ieye, W3)
    b3t = jnp.tile(b3[None, :], (1, KNN))

    gam = gamma[None, :]
    bet = beta[None, :]
    xcs, vps = [], []
    for b in range(bsz):
        qf, ktab, uf, qn = _project(vf[b], wq3t, wk3t, wu3t, r_mat)
        idxg, ssd16 = _knn(xp[b], xpt[b])
        dot16, kn16 = _sc_pass1(qf, ktab, idxg)
        attn = _mlp(qn, kn16, dot16, ssd16, w0t, k1kn, k1dot, k1dist, b1t,
                    k2, b2t, k3, b3t)
        msg = _sc_pass2(uf, idxg, attn)
        xc6, v0, v1, v2 = _tail(qf, msg, xp[b], wrq3t, wrk3t, wd3t, r_mat,
                                rt_mat, r2, r2t, gam, bet, smat, sels)
        xcs.append(xc6.reshape(n * UP, 3))
        vps.append(jnp.stack([vv.reshape(n, UP, c) for vv in (v0, v1, v2)],
                             axis=0))
    x_child = jnp.stack(xcs, axis=0)
    st = jnp.stack(vps, axis=0)
    v_child = st.reshape(bsz, 3, n * UP, c).transpose(0, 2, 3, 1)
    return x_child, v_child


# speedup vs baseline: 1.3773x; 1.0011x over previous
import dataclasses
import functools

import jax
import jax.numpy as jnp
import numpy as np
from jax import lax
from jax.experimental import pallas as pl
from jax.experimental.pallas import tpu as pltpu
from jax.experimental.pallas import tpu_sc as plsc

EPS = 1e-06
CLAMP = 50.0
KNN = 16
UP = 2
HI = lax.Precision.HIGHEST

_NW = 32
_GRP = 64


def _mm(a, b):
    return jnp.dot(a, b, precision=HI, preferred_element_type=jnp.float32)


def _mmd(a, b):
    return jnp.dot(a, b, precision=lax.Precision.DEFAULT,
                   preferred_element_type=jnp.float32)


def _sc_compiler_params():
    cp = pltpu.CompilerParams()
    if "needs_layout_passes" in pltpu.CompilerParams.__dataclass_fields__:
        cp = dataclasses.replace(cp, needs_layout_passes=False)
    if "use_tc_tiling_on_sc" in pltpu.CompilerParams.__dataclass_fields__:
        cp = dataclasses.replace(cp, use_tc_tiling_on_sc=True)
    return cp


def _proj_body(vf_ref, wq_ref, wk_ref, wu_ref, r_ref,
               qf_ref, kt_ref, uf_ref, qn_ref):
    vb = vf_ref[...]
    r = r_ref[...]
    q = _mmd(vb, wq_ref[...])
    k = _mmd(vb, wk_ref[...])
    u = _mmd(vb, wu_ref[...])
    qf_ref[...] = q
    uf_ref[...] = u
    qn_ref[...] = jnp.mean(jnp.sqrt(_mm(q * q, r)), axis=1, keepdims=True)
    kn = jnp.mean(jnp.sqrt(_mm(k * k, r)), axis=1, keepdims=True)
    tn = k.shape[0]
    pad = jnp.zeros((tn, 127), jnp.float32)
    kt_ref[...] = jnp.concatenate([k, kn, pad], axis=1)


def _project(vf, wq3t, wk3t, wu3t, r_mat, tn=512):
    m, d = vf.shape
    c = d // 3
    grid = (m // tn,)
    f32 = jnp.float32
    return pl.pallas_call(
        _proj_body,
        grid=grid,
        in_specs=[
            pl.BlockSpec((tn, d), lambda i: (i, 0)),
            pl.BlockSpec((d, d), lambda i: (0, 0)),
            pl.BlockSpec((d, d), lambda i: (0, 0)),
            pl.BlockSpec((d, d), lambda i: (0, 0)),
            pl.BlockSpec((d, c), lambda i: (0, 0)),
        ],
        out_specs=[
            pl.BlockSpec((tn, d), lambda i: (i, 0)),
            pl.BlockSpec((tn, d + 128), lambda i: (i, 0)),
            pl.BlockSpec((tn, d), lambda i: (i, 0)),
            pl.BlockSpec((tn, 1), lambda i: (i, 0)),
        ],
        out_shape=[
            jax.ShapeDtypeStruct((m, d), f32),
            jax.ShapeDtypeStruct((m, d + 128), f32),
            jax.ShapeDtypeStruct((m, d), f32),
            jax.ShapeDtypeStruct((m, 1), f32),
        ],
    )(vf, wq3t, wk3t, wu3t, r_mat)


def _knn_body(xr_ref, xct_ref, idx_ref, ssd_ref, *, n, tn):
    i = pl.program_id(0)
    xr = xr_ref[...]
    xc = xct_ref[...]
    sqr = jnp.sum(xr * xr, axis=1, keepdims=True)
    sqc = jnp.sum(xc * xc, axis=0, keepdims=True)
    dotm = _mmd(xr, xc)
    d2 = jnp.maximum(sqr + sqc - 2.0 * dotm, 0.0)
    col = lax.broadcasted_iota(jnp.int32, (tn, n), 1)
    row = i * tn + lax.broadcasted_iota(jnp.int32, (tn, n), 0)
    big = jnp.int32(0x7FFFFFFF)
    key = ((lax.bitcast_convert_type(d2, jnp.int32) + jnp.int32(0x800))
           & jnp.int32(~0xFFF)) | col
    key = jnp.where(col == row, big, key)
    cols, ssds = [], []
    for _ in range(KNN):
        kmin = jnp.min(key, axis=1, keepdims=True)
        key = jnp.where(key == kmin, big, key)
        cols.append(kmin & jnp.int32(0xFFF))
        ssds.append(lax.bitcast_convert_type(kmin & jnp.int32(~0xFFF),
                                             jnp.float32))
    idx_ref[...] = jnp.concatenate(cols, axis=1)
    ssd_ref[...] = jnp.concatenate(ssds, axis=1)


def _knn(xp, xpt, tn=256):
    _, n = xpt.shape
    grid = (n // tn,)
    body = functools.partial(_knn_body, n=n, tn=tn)
    f32 = jnp.float32
    return pl.pallas_call(
        body,
        grid=grid,
        in_specs=[
            pl.BlockSpec((tn, 8), lambda i: (i, 0)),
            pl.BlockSpec((8, n), lambda i: (0, 0)),
        ],
        out_specs=[
            pl.BlockSpec((tn, KNN), lambda i: (i, 0)),
            pl.BlockSpec((tn, KNN), lambda i: (i, 0)),
        ],
        out_shape=[
            jax.ShapeDtypeStruct((n, KNN), jnp.int32),
            jax.ShapeDtypeStruct((n, KNN), f32),
        ],
    )(xp, xpt)


def _sc1_body(q_hbm, k_hbm, idx_hbm, dot_hbm, knb_hbm,
              idx_v, q_v, kr0_v, kr1_v, dot_v, knb_v, sem0, sem1):
    c = lax.axis_index("c")
    s = lax.axis_index("s")
    wid = s * 2 + c
    per_w = q_hbm.shape[0] // _NW
    base = wid * per_w
    lanes = lax.iota(jnp.int32, 16)
    last = _GRP - 1

    def compute(p, kr_v):
        qv = [q_v[p, pl.ds(i * 16, 16)] for i in range(24)]
        dotvec = jnp.zeros((16,), jnp.float32)
        knvec = jnp.zeros((16,), jnp.float32)
        for j in range(KNN):
            acc = qv[0] * kr_v[j, pl.ds(0, 16)]
            for i in range(1, 24):
                acc = acc + qv[i] * kr_v[j, pl.ds(i * 16, 16)]
            dsc = jnp.sum(acc)
            knc = kr_v[j, pl.ds(384, 16)][0]
            onehot = lanes == j
            dotvec = jnp.where(onehot, jnp.full((16,), dsc), dotvec)
            knvec = jnp.where(onehot, jnp.full((16,), knc), knvec)
        dot_v[p, :] = dotvec
        knb_v[p, :] = knvec

    @pl.loop(0, per_w // _GRP)
    def _grp(g):
        gb = base + g * _GRP
        pltpu.sync_copy(idx_hbm.at[pl.ds(gb, _GRP)], idx_v)
        pltpu.sync_copy(q_hbm.at[pl.ds(gb, _GRP)], q_v)
        pltpu.async_copy(k_hbm.at[idx_v.at[0]], kr0_v, sem0)

        @pl.loop(0, _GRP // 2)
        def _pt(t):
            p0 = 2 * t
            pltpu.async_copy(k_hbm.at[idx_v.at[p0 + 1]], kr1_v, sem1)
            pltpu.make_async_copy(k_hbm.at[idx_v.at[p0]], kr0_v, sem0).wait()
            compute(p0, kr0_v)
            nxt = jnp.minimum(p0 + 2, last)
            pltpu.async_copy(k_hbm.at[idx_v.at[nxt]], kr0_v, sem0)
            pltpu.make_async_copy(k_hbm.at[idx_v.at[p0 + 1]], kr1_v,
                                  sem1).wait()
            compute(p0 + 1, kr1_v)

        pltpu.make_async_copy(k_hbm.at[idx_v.at[last]], kr0_v, sem0).wait()
        pltpu.sync_copy(dot_v, dot_hbm.at[pl.ds(gb, _GRP)])
        pltpu.sync_copy(knb_v, knb_hbm.at[pl.ds(gb, _GRP)])


def _sc_pass1(qf, ktab, idxg):
    m, d = qf.shape
    dk = ktab.shape[1]
    f32 = jnp.float32
    mesh = plsc.VectorSubcoreMesh(core_axis_name="c", subcore_axis_name="s")
    fn = pl.kernel(
        _sc1_body,
        mesh=mesh,
        out_type=[
            jax.ShapeDtypeStruct((m, KNN), f32),
            jax.ShapeDtypeStruct((m, KNN), f32),
        ],
        scratch_types=[
            pltpu.VMEM((_GRP, KNN), jnp.int32),
            pltpu.VMEM((_GRP, d), f32),
            pltpu.VMEM((KNN, dk), f32),
            pltpu.VMEM((KNN, dk), f32),
            pltpu.VMEM((_GRP, KNN), f32),
            pltpu.VMEM((_GRP, KNN), f32),
            pltpu.SemaphoreType.DMA,
            pltpu.SemaphoreType.DMA,
        ],
        compiler_params=_sc_compiler_params(),
    )
    return fn(qf, ktab, idxg)


def _mlp_body(qn_ref, knb_ref, dot_ref, ssd_ref,
              w1_ref, b1_ref, w2_ref, b2_ref, w3_ref, b3_ref, attn_ref):
    qn = qn_ref[...]
    knb = knb_ref[...]
    dot = dot_ref[...] * (1.0 / 128.0)
    dist = jnp.sqrt(ssd_ref[...])
    w1 = w1_ref[...]
    b1 = b1_ref[...]
    w2 = w2_ref[...]
    b2 = b2_ref[...]
    w3 = w3_ref[...]
    b3 = b3_ref[...]

    def silu(t):
        return t * (1.0 / (1.0 + jnp.exp(-t)))

    ls = []
    for j in range(KNN):
        ej = jnp.concatenate(
            [qn, knb[:, j:j + 1], dot[:, j:j + 1], dist[:, j:j + 1]], axis=1)
        h1 = silu(_mmd(ej, w1) + b1)
        h2 = silu(_mmd(h1, w2) + b2)
        ls.append(_mmd(h2, w3) + b3)
    logits = jnp.concatenate(ls, axis=1)
    logits = jnp.clip(logits, -10.0, 10.0)
    mx = jnp.max(logits, axis=1, keepdims=True)
    e = jnp.exp(logits - mx)
    attn_ref[...] = e / jnp.sum(e, axis=1, keepdims=True)


def _mlp(qn, knb, dot16, ssd16, w1, b1t, w2, b2t, w3, b3t, tn=512):
    m = qn.shape[0]
    h = w2.shape[0]
    grid = (m // tn,)
    return pl.pallas_call(
        _mlp_body,
        grid=grid,
        in_specs=[
            pl.BlockSpec((tn, 1), lambda i: (i, 0)),
            pl.BlockSpec((tn, KNN), lambda i: (i, 0)),
            pl.BlockSpec((tn, KNN), lambda i: (i, 0)),
            pl.BlockSpec((tn, KNN), lambda i: (i, 0)),
            pl.BlockSpec((4, h), lambda i: (0, 0)),
            pl.BlockSpec((1, h), lambda i: (0, 0)),
            pl.BlockSpec((h, h), lambda i: (0, 0)),
            pl.BlockSpec((1, h), lambda i: (0, 0)),
            pl.BlockSpec((h, 1), lambda i: (0, 0)),
            pl.BlockSpec((1, 1), lambda i: (0, 0)),
        ],
        out_specs=pl.BlockSpec((tn, KNN), lambda i: (i, 0)),
        out_shape=jax.ShapeDtypeStruct((m, KNN), jnp.float32),
    )(qn, knb, dot16, ssd16, w1, b1t, w2, b2t, w3, b3t)


def _sc2_body(u_hbm, idx_hbm, attn_hbm, msg_hbm,
              idx_v, attn_v, ur0_v, ur1_v, msg_v, sem0, sem1):
    c = lax.axis_index("c")
    s = lax.axis_index("s")
    wid = s * 2 + c
    per_w = msg_hbm.shape[0] // _NW
    base = wid * per_w
    last = _GRP - 1

    def compute(p, ur_v):
        arow = attn_v[p, :]
        accs = [jnp.zeros((16,), jnp.float32) for _ in range(24)]
        for j in range(KNN):
            wv = jnp.full((16,), arow[j])
            for i in range(24):
                accs[i] = accs[i] + wv * ur_v[j, pl.ds(i * 16, 16)]
        for i in range(24):
            msg_v[p, pl.ds(i * 16, 16)] = accs[i]

    @pl.loop(0, per_w // _GRP)
    def _grp(g):
        gb = base + g * _GRP
        pltpu.sync_copy(idx_hbm.at[pl.ds(gb, _GRP)], idx_v)
        pltpu.sync_copy(attn_hbm.at[pl.ds(gb, _GRP)], attn_v)
        pltpu.async_copy(u_hbm.at[idx_v.at[0]], ur0_v, sem0)

        @pl.loop(0, _GRP // 2)
        def _pt(t):
            p0 = 2 * t
            pltpu.async_copy(u_hbm.at[idx_v.at[p0 + 1]], ur1_v, sem1)
            pltpu.make_async_copy(u_hbm.at[idx_v.at[p0]], ur0_v, sem0).wait()
            compute(p0, ur0_v)
            nxt = jnp.minimum(p0 + 2, last)
            pltpu.async_copy(u_hbm.at[idx_v.at[nxt]], ur0_v, sem0)
            pltpu.make_async_copy(u_hbm.at[idx_v.at[p0 + 1]], ur1_v,
                                  sem1).wait()
            compute(p0 + 1, ur1_v)

        pltpu.make_async_copy(u_hbm.at[idx_v.at[last]], ur0_v, sem0).wait()
        pltpu.sync_copy(msg_v, msg_hbm.at[pl.ds(gb, _GRP)])


def _sc_pass2(uf, idxg, attn):
    m, d = uf.shape
    f32 = jnp.float32
    mesh = plsc.VectorSubcoreMesh(core_axis_name="c", subcore_axis_name="s")
    fn = pl.kernel(
        _sc2_body,
        mesh=mesh,
        out_type=jax.ShapeDtypeStruct((m, d), f32),
        scratch_types=[
            pltpu.VMEM((_GRP, KNN), jnp.int32),
            pltpu.VMEM((_GRP, KNN), f32),
            pltpu.VMEM((KNN, d), f32),
            pltpu.VMEM((KNN, d), f32),
            pltpu.VMEM((_GRP, d), f32),
            pltpu.SemaphoreType.DMA,
            pltpu.SemaphoreType.DMA,
        ],
        compiler_params=_sc_compiler_params(),
    )
    return fn(uf, idxg, attn)


def _tail_body(qf_ref, msg_ref, xp_ref,
               wrq_ref, wd_ref, r_ref, rt_ref, r2_ref, r2t_ref,
               gam_ref, bet_ref, s_ref, sel0_ref,
               xc_ref, v0_ref, v1_ref, v2_ref):
    r = r_ref[...]
    rt = rt_ref[...]
    out = qf_ref[...] + 0.5 * msg_ref[...]
    norm = jnp.maximum(jnp.sqrt(_mm(out * out, r)), EPS)
    mean = jnp.mean(norm, axis=1, keepdims=True)
    dev = norm - mean
    nchan = norm.shape[1]
    var = jnp.sum(dev * dev, axis=1, keepdims=True) * (1.0 / (nchan - 1))
    std = jnp.maximum(jnp.sqrt(var), EPS)
    ns = (dev / std) * gam_ref[...] + bet_ref[...]
    fac = jnp.maximum(ns, EPS) / norm
    out = out * _mm(fac, rt)
    n2 = jnp.maximum(jnp.sqrt(_mm(out * out, r)), EPS)
    out = out * _mm(jnp.minimum(CLAMP / n2, 1.0), rt)
    d = out.shape[1]
    qkk = _mmd(out, wrq_ref[...])
    q = qkk[:, :d]
    kk = qkk[:, d:]
    dqk = _mm(q * kk, r)
    kn2 = jnp.maximum(_mm(kk * kk, r), EPS * EPS)
    t = jnp.where(dqk >= 0.0, 0.0, dqk / kn2)
    h = q - _mm(t, rt) * kk
    disp = _mmd(h, wd_ref[...])
    dn = jnp.maximum(jnp.sqrt(_mm(disp * disp, r2_ref[...])), EPS)
    scl = jnp.tanh(dn) / dn * 0.1
    disp = disp * _mm(scl, r2t_ref[...])
    xc_ref[...] = _mm(xp_ref[...], s_ref[...]) + disp
    c = d // 3
    hperm = _mm(h, sel0_ref[...])
    h0 = hperm[:, :c]
    h1v = hperm[:, c:2 * c]
    h2v = hperm[:, 2 * c:]
    v0_ref[...] = jnp.concatenate([h0, h0], axis=1)
    v1_ref[...] = jnp.concatenate([h1v, h1v], axis=1)
    v2_ref[...] = jnp.concatenate([h2v, h2v], axis=1)


def _tail(qf, msg, xp, wrcat, wd3t, r_mat, rt_mat, r2, r2t, gam, bet,
          smat, selcat, tn=256):
    m, d = qf.shape
    c = d // 3
    grid = (m // tn,)
    f32 = jnp.float32
    return pl.pallas_call(
        _tail_body,
        grid=grid,
        in_specs=[
            pl.BlockSpec((tn, d), lambda i: (i, 0)),
            pl.BlockSpec((tn, d), lambda i: (i, 0)),
            pl.BlockSpec((tn, 8), lambda i: (i, 0)),
            pl.BlockSpec((d, 2 * d), lambda i: (0, 0)),
            pl.BlockSpec((d, 3 * UP), lambda i: (0, 0)),
            pl.BlockSpec((d, c), lambda i: (0, 0)),
            pl.BlockSpec((c, d), lambda i: (0, 0)),
            pl.BlockSpec((3 * UP, UP), lambda i: (0, 0)),
            pl.BlockSpec((UP, 3 * UP), lambda i: (0, 0)),
            pl.BlockSpec((1, c), lambda i: (0, 0)),
            pl.BlockSpec((1, c), lambda i: (0, 0)),
            pl.BlockSpec((8, 3 * UP), lambda i: (0, 0)),
            pl.BlockSpec((d, d), lambda i: (0, 0)),
        ],
        out_specs=[
            pl.BlockSpec((tn, 3 * UP), lambda i: (i, 0)),
            pl.BlockSpec((tn, 2 * c), lambda i: (i, 0)),
            pl.BlockSpec((tn, 2 * c), lambda i: (i, 0)),
            pl.BlockSpec((tn, 2 * c), lambda i: (i, 0)),
        ],
        out_shape=[
            jax.ShapeDtypeStruct((m, 3 * UP), f32),
            jax.ShapeDtypeStruct((m, 2 * c), f32),
            jax.ShapeDtypeStruct((m, 2 * c), f32),
            jax.ShapeDtypeStruct((m, 2 * c), f32),
        ],
    )(qf, msg, xp, wrcat, wd3t, r_mat, rt_mat, r2, r2t, gam, bet, smat,
      selcat)


def kernel(x, v, Wq, Wk, Wu, W1, b1, W2, b2, W3, b3, gamma, beta, Wrq, Wrk,
           Wd):
    bsz, n, c, _ = v.shape
    m = bsz * n
    d = 3 * c
    f32 = jnp.float32
    i3 = jnp.eye(3, dtype=f32)

    vf = v.reshape(bsz, n, d)
    xp = jnp.concatenate([x, jnp.zeros((bsz, n, 5), f32)], axis=2)
    xpt = jnp.pad(jnp.swapaxes(x, 1, 2), ((0, 0), (0, 5), (0, 0)))

    wq3t = jnp.kron(Wq, i3).T
    wk3t = jnp.kron(Wk, i3).T
    wu3t = jnp.kron(Wu, i3).T
    wrcat = jnp.concatenate([jnp.kron(Wrq, i3).T, jnp.kron(Wrk, i3).T],
                            axis=1)
    wd3t = jnp.kron(Wd, i3).T
    r_np = np.kron(np.eye(c, dtype=np.float32), np.ones((3, 1), np.float32))
    r_mat = jnp.asarray(r_np)
    rt_mat = jnp.asarray(r_np.T)
    r2_np = np.kron(np.eye(UP, dtype=np.float32), np.ones((3, 1), np.float32))
    r2 = jnp.asarray(r2_np)
    r2t = jnp.asarray(r2_np.T)
    s_np = np.zeros((8, 3 * UP), np.float32)
    for u in range(UP):
        for dd in range(3):
            s_np[dd, u * 3 + dd] = 1.0
    smat = jnp.asarray(s_np)
    sel_np = np.concatenate(
        [np.kron(np.eye(c, dtype=np.float32),
                 np.eye(3, dtype=np.float32)[:, dd:dd + 1])
         for dd in range(3)], axis=1)
    selcat = jnp.asarray(sel_np)

    b1t = b1[None, :]
    b2t = b2[None, :]
    b3t = b3[None, :]

    gam = gamma[None, :]
    bet = beta[None, :]
    xcs, vps = [], []
    for b in range(bsz):
        qf, ktab, uf, qn = _project(vf[b], wq3t, wk3t, wu3t, r_mat)
        idxg, ssd16 = _knn(xp[b], xpt[b])
        dot16, kn16 = _sc_pass1(qf, ktab, idxg)
        attn = _mlp(qn, kn16, dot16, ssd16, W1, b1t, W2, b2t, W3, b3t)
        msg = _sc_pass2(uf, idxg, attn)
        xc6, v0, v1, v2 = _tail(qf, msg, xp[b], wrcat, wd3t, r_mat,
                                rt_mat, r2, r2t, gam, bet, smat, selcat)
        xcs.append(xc6.reshape(n * UP, 3))
        vps.append(jnp.stack([vv.reshape(n, UP, c) for vv in (v0, v1, v2)],
                             axis=0))
    x_child = jnp.stack(xcs, axis=0)
    st = jnp.stack(vps, axis=0)
    v_child = st.reshape(bsz, 3, n * UP, c).transpose(0, 2, 3, 1)
    return x_child, v_child
